# Initial kernel scaffold; baseline (speedup 1.0000x reference)
import functools

import jax
import jax.numpy as jnp
import numpy as np
from jax import lax
from jax.experimental import pallas as pl
from jax.experimental.pallas import tpu as pltpu
from jax.experimental.pallas import tpu_sc as plsc

_N = 50000
_E = 800000
_C = 16
_NS = 0.2

_HALF = 8
_ROW = 48
_ACCW = 32
_B = 128
_NSUB = 16
_BN = 6250
_NB = _N // _BN

_HI = jax.lax.Precision.HIGHEST


def _tables0_body(pos_ref, ms_ref, md_ref, src_ref, dst_ref):
    p = pos_ref[...]
    for h in range(2):
        src_ref[h] = lax.dot(p, ms_ref[:, h * 48:(h + 1) * 48],
                             precision=_HI, preferred_element_type=jnp.float32)
        dst_ref[h] = lax.dot(p, md_ref[:, h * 48:(h + 1) * 48],
                             precision=_HI, preferred_element_type=jnp.float32)


def _tables0(pos, m0s, m0d):
    return pl.pallas_call(
        _tables0_body,
        grid=(_NB,),
        in_specs=[
            pl.BlockSpec((_BN, 3), lambda i: (i, 0)),
            pl.BlockSpec((3, 96), lambda i: (0, 0)),
            pl.BlockSpec((3, 96), lambda i: (0, 0)),
        ],
        out_specs=[
            pl.BlockSpec((2, _BN, 48), lambda i: (0, i, 0)),
            pl.BlockSpec((2, _BN, 48), lambda i: (0, i, 0)),
        ],
        out_shape=[
            jax.ShapeDtypeStruct((2, _N, 48), jnp.float32),
            jax.ShapeDtypeStruct((2, _N, 48), jnp.float32),
        ],
    )(pos, m0s, m0d)


def _mean_x(acc_ref):
    a0 = acc_ref[0]
    a1 = acc_ref[1]
    cnt = jnp.maximum(a0[:, 24:25], 1.0)
    return jnp.concatenate([a0[:, :24], a1[:, :24]], axis=1) / cnt


def _stats_body(acc_ref, s_ref, sum_ref, sq_ref):
    x = _mean_x(acc_ref)
    nsq = lax.dot(x * x, s_ref[...], precision=_HI,
                  preferred_element_type=jnp.float32)
    norm = jnp.sqrt(nsq + 1e-12)
    sum_ref[0] = jnp.sum(norm, axis=0)
    sq_ref[0] = jnp.sum(norm * norm, axis=0)


def _stats(acc, s_mat):
    return pl.pallas_call(
        _stats_body,
        grid=(_NB,),
        in_specs=[
            pl.BlockSpec((2, _BN, _ACCW), lambda i: (0, i, 0)),
            pl.BlockSpec((48, 16), lambda i: (0, 0)),
        ],
        out_specs=[
            pl.BlockSpec((1, 16), lambda i: (i, 0)),
            pl.BlockSpec((1, 16), lambda i: (i, 0)),
        ],
        out_shape=[
            jax.ShapeDtypeStruct((_NB, 16), jnp.float32),
            jax.ShapeDtypeStruct((_NB, 16), jnp.float32),
        ],
    )(acc, s_mat)


def _bn_x(acc_ref, mv_ref, g_ref, s_ref, st_ref):
    x = _mean_x(acc_ref)
    nsq = lax.dot(x * x, s_ref[...], precision=_HI,
                  preferred_element_type=jnp.float32)
    norm = jnp.sqrt(nsq + 1e-12)
    mean = mv_ref[0:1, :]
    var = mv_ref[1:2, :]
    norm_bn = g_ref[0:1, :] * (norm - mean) / jnp.sqrt(var + 1e-5)
    s16 = norm_bn / (norm + 1e-5)
    s48 = lax.dot(s16, st_ref[...], precision=_HI,
                  preferred_element_type=jnp.float32)
    return x * s48


def _tables_body(acc_ref, mv_ref, g_ref, s_ref, st_ref, ms_ref, md_ref,
                 src_ref, dst_ref):
    xb = _bn_x(acc_ref, mv_ref, g_ref, s_ref, st_ref)
    for h in range(2):
        src_ref[h] = lax.dot(xb, ms_ref[:, h * 48:(h + 1) * 48],
                             precision=_HI, preferred_element_type=jnp.float32)
        dst_ref[h] = lax.dot(xb, md_ref[:, h * 48:(h + 1) * 48],
                             precision=_HI, preferred_element_type=jnp.float32)


def _tables(acc, mv, gamma, s_mat, st_mat, ms, md):
    return pl.pallas_call(
        _tables_body,
        grid=(_NB,),
        in_specs=[
            pl.BlockSpec((2, _BN, _ACCW), lambda i: (0, i, 0)),
            pl.BlockSpec((2, 16), lambda i: (0, 0)),
            pl.BlockSpec((1, 16), lambda i: (0, 0)),
            pl.BlockSpec((48, 16), lambda i: (0, 0)),
            pl.BlockSpec((16, 48), lambda i: (0, 0)),
            pl.BlockSpec((48, 96), lambda i: (0, 0)),
            pl.BlockSpec((48, 96), lambda i: (0, 0)),
        ],
        out_specs=[
            pl.BlockSpec((2, _BN, 48), lambda i: (0, i, 0)),
            pl.BlockSpec((2, _BN, 48), lambda i: (0, i, 0)),
        ],
        out_shape=[
            jax.ShapeDtypeStruct((2, _N, 48), jnp.float32),
            jax.ShapeDtypeStruct((2, _N, 48), jnp.float32),
        ],
    )(acc, mv, gamma, s_mat, st_mat, ms, md)


def _readout_body(acc_ref, mv_ref, g_ref, s_ref, st_ref, out_ref):
    xb = _bn_x(acc_ref, mv_ref, g_ref, s_ref, st_ref)
    nsq = lax.dot(xb * xb, s_ref[...], precision=_HI,
                  preferred_element_type=jnp.float32)
    out_ref[...] = jnp.sqrt(nsq + 1e-12)


def _readout(acc, mv, gamma, s_mat, st_mat):
    return pl.pallas_call(
        _readout_body,
        grid=(_NB,),
        in_specs=[
            pl.BlockSpec((2, _BN, _ACCW), lambda i: (0, i, 0)),
            pl.BlockSpec((2, 16), lambda i: (0, 0)),
            pl.BlockSpec((1, 16), lambda i: (0, 0)),
            pl.BlockSpec((48, 16), lambda i: (0, 0)),
            pl.BlockSpec((16, 48), lambda i: (0, 0)),
        ],
        out_specs=pl.BlockSpec((_BN, 16), lambda i: (i, 0)),
        out_shape=jax.ShapeDtypeStruct((_N, 16), jnp.float32),
    )(acc, mv, gamma, s_mat, st_mat)


def _sc_body(src_hbm, dst_hbm, ei_hbm, z_hbm, out_hbm,
             isrc, idst, idstt, rs, rd, ov, acc, sem1, sem2):
    cid = lax.axis_index("c")
    sid = lax.axis_index("s")

    chunk = _N // _NSUB
    r0 = sid * chunk
    pltpu.sync_copy(z_hbm.at[pl.ds(r0, chunk)], acc.at[pl.ds(r0, chunk)])
    plsc.subcore_barrier()

    coff = cid * _N
    iota16 = lax.iota(jnp.int32, 16)
    ones16 = jnp.full((16,), 1.0, jnp.float32)

    ntiles = _E // _B
    lo = ntiles // _NSUB
    extra = ntiles - lo * _NSUB
    nt = jnp.where(sid < extra, lo + 1, lo)
    t0 = jnp.where(sid < extra, (lo + 1) * sid,
                   extra * (lo + 1) + lo * (sid - extra))

    def tile_body(t, carry):
        off = (t0 + t) * _B
        pltpu.sync_copy(ei_hbm.at[0, pl.ds(off, _B)], isrc)
        pltpu.sync_copy(ei_hbm.at[1, pl.ds(off, _B)], idst)
        for u in range(_B // 16):
            sl = pl.ds(u * 16, 16)
            isrc[sl] = isrc[sl] + coff
            idstt[sl] = idst[sl] + coff
        h1 = pltpu.async_copy(src_hbm.at[isrc], rs, sem1)
        h2 = pltpu.async_copy(dst_hbm.at[idstt], rd, sem2)
        h1.wait()
        h2.wait()
        for g in range(_B // 16):
            rows = iota16 + g * 16
            mv = []
            dv = []
            for j in range(24):
                cj = jnp.full((16,), j, jnp.int32)
                mv.append(plsc.load_gather(rs, [rows, cj]) +
                          plsc.load_gather(rd, [rows, cj]))
            for j in range(24):
                cj = jnp.full((16,), 24 + j, jnp.int32)
                dv.append(plsc.load_gather(rs, [rows, cj]) +
                          plsc.load_gather(rd, [rows, cj]))
            for c in range(_HALF):
                b3 = 3 * c
                mx, my, mz = mv[b3], mv[b3 + 1], mv[b3 + 2]
                dx, dy, dz = dv[b3], dv[b3 + 1], dv[b3 + 2]
                dot = mx * dx + my * dy + mz * dz
                dsq = dx * dx + dy * dy + dz * dz + 1e-6
                coef = jnp.where(dot < 0.0, (1.0 - _NS) * (dot / dsq), 0.0)
                plsc.store_scatter(ov, [rows, jnp.full((16,), b3, jnp.int32)],
                                   mx - coef * dx)
                plsc.store_scatter(ov, [rows, jnp.full((16,), b3 + 1, jnp.int32)],
                                   my - coef * dy)
                plsc.store_scatter(ov, [rows, jnp.full((16,), b3 + 2, jnp.int32)],
                                   mz - coef * dz)
            plsc.store_scatter(ov, [rows, jnp.full((16,), 24, jnp.int32)], ones16)
        pltpu.sync_copy(ov, acc.at[idst], add=True)
        return carry

    lax.fori_loop(0, nt, tile_body, 0)

    plsc.subcore_barrier()
    pltpu.sync_copy(acc.at[pl.ds(r0, chunk)], out_hbm.at[cid, pl.ds(r0, chunk)])


def _sc_edge_pass(src_tab, dst_tab, edge_index, zeros):
    mesh = plsc.VectorSubcoreMesh(core_axis_name="c", subcore_axis_name="s")
    k = functools.partial(
        pl.kernel,
        mesh=mesh,
        out_type=jax.ShapeDtypeStruct((2, _N, _ACCW), jnp.float32),
        scratch_types=[
            pltpu.VMEM((_B,), jnp.int32),
            pltpu.VMEM((_B,), jnp.int32),
            pltpu.VMEM((_B,), jnp.int32),
            pltpu.VMEM((_B, _ROW), jnp.float32),
            pltpu.VMEM((_B, _ROW), jnp.float32),
            pltpu.VMEM((_B, _ACCW), jnp.float32),
            pltpu.VMEM_SHARED((_N, _ACCW), jnp.float32),
            pltpu.SemaphoreType.DMA,
            pltpu.SemaphoreType.DMA,
        ],
    )(_sc_body)
    return k(src_tab, dst_tab, edge_index, zeros)


def _part(mat):
    return jnp.kron(mat.T, jnp.eye(3, dtype=jnp.float32))


def _fold(w_msg, u_dir):
    w1 = w_msg[:, :_C]
    g2 = w_msg[:, _C:] - w1
    ug1 = u_dir @ w1
    ug2 = u_dir @ g2
    msrc = jnp.concatenate(
        [_part(w1[:_HALF]), _part(ug1[:_HALF]),
         _part(w1[_HALF:]), _part(ug1[_HALF:])], axis=1)
    mdst = jnp.concatenate(
        [_part(g2[:_HALF]), _part(ug2[:_HALF]),
         _part(g2[_HALF:]), _part(ug2[_HALF:])], axis=1)
    return msrc, mdst


def _finish_stats(stats):
    sums, sqs = stats
    mean = jnp.sum(sums, axis=0) / _N
    ex2 = jnp.sum(sqs, axis=0) / _N
    var = ex2 - mean * mean
    return jnp.stack([mean, var])


def kernel(pos, edge_index, W_lift, W_msg_0, U_dir_0, gamma_0,
           W_msg_1, U_dir_1, gamma_1):
    f32 = jnp.float32
    s_mat = jnp.asarray(np.kron(np.eye(16), np.ones((3, 1))), f32)
    st_mat = s_mat.T
    zeros = jnp.zeros((_N, _ACCW), f32)

    ms0, md0 = _fold(W_msg_0, U_dir_0)
    lift = jnp.kron(W_lift.reshape(1, _C), jnp.eye(3, dtype=f32))
    src0, dst0 = _tables0(pos, lift @ ms0, lift @ md0)

    acc_a = _sc_edge_pass(src0.reshape(2 * _N, 48), dst0.reshape(2 * _N, 48),
                          edge_index, zeros)

    mv_a = _finish_stats(_stats(acc_a, s_mat))
    ms1, md1 = _fold(W_msg_1, U_dir_1)
    src1, dst1 = _tables(acc_a, mv_a, gamma_0.reshape(1, _C),
                         s_mat, st_mat, ms1, md1)

    acc_b = _sc_edge_pass(src1.reshape(2 * _N, 48), dst1.reshape(2 * _N, 48),
                          edge_index, zeros)

    mv_b = _finish_stats(_stats(acc_b, s_mat))
    return _readout(acc_b, mv_b, gamma_1.reshape(1, _C), s_mat, st_mat)


# speedup vs baseline: 8.6193x; 8.6193x over previous
import functools

import jax
import jax.numpy as jnp
import numpy as np
from jax import lax
from jax.experimental import pallas as pl
from jax.experimental.pallas import tpu as pltpu
from jax.experimental.pallas import tpu_sc as plsc

_N = 50000
_E = 800000
_C = 16
_NS = 0.2

_HALF = 8
_ROW = 48
_ACCW = 32
_B = 80
_NSUB = 16
_BN = 5000
_NB = _N // _BN

_HI = jax.lax.Precision.HIGHEST


def _qtab(x, pq_ref):
    xb = x.astype(jnp.bfloat16).astype(jnp.float32)
    return lax.dot(xb, pq_ref[...], precision=_HI,
                   preferred_element_type=jnp.float32)


def _tables0_body(pos_ref, lift_ref, pq_ref, src_ref, dst_ref):
    x = lax.dot(pos_ref[...], lift_ref[...], precision=_HI,
                preferred_element_type=jnp.float32)
    src_ref[...] = x
    dst_ref[:, :48] = x
    dst_ref[:, 48:] = _qtab(x, pq_ref)


def _tables0(pos, lift, pq):
    return pl.pallas_call(
        _tables0_body,
        grid=(_NB,),
        in_specs=[
            pl.BlockSpec((_BN, 3), lambda i: (i, 0)),
            pl.BlockSpec((3, 48), lambda i: (0, 0)),
            pl.BlockSpec((48, 48), lambda i: (0, 0)),
        ],
        out_specs=[
            pl.BlockSpec((_BN, 48), lambda i: (i, 0)),
            pl.BlockSpec((_BN, 96), lambda i: (i, 0)),
        ],
        out_shape=[
            jax.ShapeDtypeStruct((_N, 48), jnp.float32),
            jax.ShapeDtypeStruct((_N, 96), jnp.float32),
        ],
    )(pos, lift, pq)


def _mean_x(acc_ref):
    a0 = acc_ref[0]
    a1 = acc_ref[1]
    cnt = jnp.maximum(a0[:, 24:25], 1.0)
    return jnp.concatenate([a0[:, :24], a1[:, :24]], axis=1) / cnt


def _stats_body(acc_ref, s_ref, st_ref):
    x = _mean_x(acc_ref)
    nsq = lax.dot(x * x, s_ref[...], precision=_HI,
                  preferred_element_type=jnp.float32)
    norm = jnp.sqrt(nsq + 1e-12)
    st_ref[0, 0] = jnp.sum(norm, axis=0)
    st_ref[0, 1] = jnp.sum(norm * norm, axis=0)


def _stats(acc, s_mat):
    return pl.pallas_call(
        _stats_body,
        grid=(_NB,),
        in_specs=[
            pl.BlockSpec((2, _BN, _ACCW), lambda i: (0, i, 0)),
            pl.BlockSpec((48, 16), lambda i: (0, 0)),
        ],
        out_specs=pl.BlockSpec((1, 8, 16), lambda i: (i, 0, 0)),
        out_shape=jax.ShapeDtypeStruct((_NB, 8, 16), jnp.float32),
    )(acc, s_mat)


def _bn_x(acc_ref, mv_ref, g_ref, s_ref, st_ref):
    x = _mean_x(acc_ref)
    nsq = lax.dot(x * x, s_ref[...], precision=_HI,
                  preferred_element_type=jnp.float32)
    norm = jnp.sqrt(nsq + 1e-12)
    mean = mv_ref[0:1, :]
    var = mv_ref[1:2, :]
    norm_bn = g_ref[0:1, :] * (norm - mean) / jnp.sqrt(var + 1e-5)
    s16 = norm_bn / (norm + 1e-5)
    s48 = lax.dot(s16, st_ref[...], precision=_HI,
                  preferred_element_type=jnp.float32)
    return x * s48


def _tables_body(acc_ref, mv_ref, g_ref, s_ref, st_ref, pq_ref,
                 src_ref, dst_ref):
    x = _bn_x(acc_ref, mv_ref, g_ref, s_ref, st_ref)
    src_ref[...] = x
    dst_ref[:, :48] = x
    dst_ref[:, 48:] = _qtab(x, pq_ref)


def _tables(acc, mv, gamma, s_mat, st_mat, pq):
    return pl.pallas_call(
        _tables_body,
        grid=(_NB,),
        in_specs=[
            pl.BlockSpec((2, _BN, _ACCW), lambda i: (0, i, 0)),
            pl.BlockSpec((2, 16), lambda i: (0, 0)),
            pl.BlockSpec((1, 16), lambda i: (0, 0)),
            pl.BlockSpec((48, 16), lambda i: (0, 0)),
            pl.BlockSpec((16, 48), lambda i: (0, 0)),
            pl.BlockSpec((48, 48), lambda i: (0, 0)),
        ],
        out_specs=[
            pl.BlockSpec((_BN, 48), lambda i: (i, 0)),
            pl.BlockSpec((_BN, 96), lambda i: (i, 0)),
        ],
        out_shape=[
            jax.ShapeDtypeStruct((_N, 48), jnp.float32),
            jax.ShapeDtypeStruct((_N, 96), jnp.float32),
        ],
    )(acc, mv, gamma, s_mat, st_mat, pq)


def _readout_body(acc_ref, mv_ref, g_ref, s_ref, st_ref, out_ref):
    xb = _bn_x(acc_ref, mv_ref, g_ref, s_ref, st_ref)
    nsq = lax.dot(xb * xb, s_ref[...], precision=_HI,
                  preferred_element_type=jnp.float32)
    out_ref[...] = jnp.sqrt(nsq + 1e-12)


def _readout(acc, mv, gamma, s_mat, st_mat):
    return pl.pallas_call(
        _readout_body,
        grid=(_NB,),
        in_specs=[
            pl.BlockSpec((2, _BN, _ACCW), lambda i: (0, i, 0)),
            pl.BlockSpec((2, 16), lambda i: (0, 0)),
            pl.BlockSpec((1, 16), lambda i: (0, 0)),
            pl.BlockSpec((48, 16), lambda i: (0, 0)),
            pl.BlockSpec((16, 48), lambda i: (0, 0)),
        ],
        out_specs=pl.BlockSpec((_BN, 16), lambda i: (i, 0)),
        out_shape=jax.ShapeDtypeStruct((_N, 16), jnp.float32),
    )(acc, mv, gamma, s_mat, st_mat)


def _rnd_bf16(v):
    u = plsc.bitcast(v, jnp.int32)
    r = (u + 32767 + ((u >> 16) & 1)) & (-65536)
    return plsc.bitcast(r, jnp.float32)


def _sc_body(src_hbm, dst_hbm, w1_hbm, u_hbm, ei_hbm, z_hbm, out_hbm,
             isrc, idst, rs, rd, ov, wv1, wvu, msc, acc, sem1, sem2):
    cid = lax.axis_index("c")
    sid = lax.axis_index("s")

    pltpu.sync_copy(w1_hbm, wv1)
    pltpu.sync_copy(u_hbm.at[cid], wvu)

    c1 = 3128
    clast = _N - 15 * c1

    @pl.when(sid < _NSUB - 1)
    def _():
        pltpu.sync_copy(z_hbm.at[pl.ds(sid * c1, c1)],
                        acc.at[pl.ds(sid * c1, c1)])

    @pl.when(sid == _NSUB - 1)
    def _():
        pltpu.sync_copy(z_hbm.at[pl.ds(15 * c1, clast)],
                        acc.at[pl.ds(15 * c1, clast)])

    plsc.subcore_barrier()

    iota16 = lax.iota(jnp.int32, 16)
    ones16 = jnp.full((16,), 1.0, jnp.float32)

    ntiles = _E // _B
    lo = ntiles // _NSUB
    extra = ntiles - lo * _NSUB
    nt = jnp.where(sid < extra, lo + 1, lo)
    t0 = jnp.where(sid < extra, (lo + 1) * sid,
                   extra * (lo + 1) + lo * (sid - extra))
    mrow0 = 48 + cid * 24

    def tile_body(t, carry):
        off = (t0 + t) * _B
        pltpu.sync_copy(ei_hbm.at[0, pl.ds(off, _B)], isrc)
        pltpu.sync_copy(ei_hbm.at[1, pl.ds(off, _B)], idst)
        h1 = pltpu.async_copy(src_hbm.at[isrc], rs, sem1)
        h2 = pltpu.async_copy(dst_hbm.at[idst], rd, sem2)
        h1.wait()
        h2.wait()

        def group(g, c2):
            rows = iota16 + g * 16
            for j in range(48):
                cj = jnp.full((16,), j, jnp.int32)
                msc[96 + j, :] = _rnd_bf16(plsc.load_gather(rs, [rows, cj]) -
                                           plsc.load_gather(rd, [rows, cj]))
            for o0 in range(0, 16, 4):
                mo = [plsc.load_gather(
                          rd, [rows, jnp.full((16,), 48 + (o0 + oc) * 3 + dd,
                                              jnp.int32)])
                      for oc in range(4) for dd in range(3)]
                for k in range(16):
                    yk = [msc[96 + k * 3 + dd, :] for dd in range(3)]
                    for oc in range(4):
                        w = wv1[(o0 + oc) * 16 + k, :]
                        for dd in range(3):
                            mo[oc * 3 + dd] = mo[oc * 3 + dd] + w * yk[dd]
                for oc in range(4):
                    for dd in range(3):
                        j = (o0 + oc) * 3 + dd
                        msc[48 + j, :] = mo[oc * 3 + dd]
                        msc[j, :] = _rnd_bf16(mo[oc * 3 + dd])
            for o0 in range(0, 8, 4):
                dacc = [jnp.zeros((16,), jnp.float32) for _ in range(12)]
                for k in range(16):
                    mb = [msc[k * 3 + dd, :] for dd in range(3)]
                    for oc in range(4):
                        u = wvu[(o0 + oc) * 16 + k, :]
                        for dd in range(3):
                            dacc[oc * 3 + dd] = dacc[oc * 3 + dd] + u * mb[dd]
                for oc in range(4):
                    c = o0 + oc
                    b3 = 3 * c
                    mrow = mrow0 + b3
                    mx = plsc.load_gather(msc, [jnp.full((16,), mrow, jnp.int32),
                                                iota16])
                    my = plsc.load_gather(msc, [jnp.full((16,), mrow + 1,
                                                         jnp.int32), iota16])
                    mz = plsc.load_gather(msc, [jnp.full((16,), mrow + 2,
                                                         jnp.int32), iota16])
                    dx, dy, dz = dacc[oc * 3], dacc[oc * 3 + 1], dacc[oc * 3 + 2]
                    dot = mx * dx + my * dy + mz * dz
                    dsq = dx * dx + dy * dy + dz * dz + 1e-6
                    coef = jnp.where(dot < 0.0, (1.0 - _NS) * (dot / dsq), 0.0)
                    plsc.store_scatter(ov, [rows, jnp.full((16,), b3, jnp.int32)],
                                       mx - coef * dx)
                    plsc.store_scatter(ov, [rows, jnp.full((16,), b3 + 1,
                                                           jnp.int32)],
                                       my - coef * dy)
                    plsc.store_scatter(ov, [rows, jnp.full((16,), b3 + 2,
                                                           jnp.int32)],
                                       mz - coef * dz)
            plsc.store_scatter(ov, [rows, jnp.full((16,), 24, jnp.int32)], ones16)
            return c2

        lax.fori_loop(0, _B // 16, group, 0)
        pltpu.sync_copy(ov, acc.at[idst], add=True)
        return carry

    lax.fori_loop(0, nt, tile_body, 0)

    plsc.subcore_barrier()

    @pl.when(sid < _NSUB - 1)
    def _():
        pltpu.sync_copy(acc.at[pl.ds(sid * c1, c1)],
                        out_hbm.at[cid, pl.ds(sid * c1, c1)])

    @pl.when(sid == _NSUB - 1)
    def _():
        pltpu.sync_copy(acc.at[pl.ds(15 * c1, clast)],
                        out_hbm.at[cid, pl.ds(15 * c1, clast)])


def _sc_edge_pass(src_tab, dst_tab, w1s, us, edge_index, zeros):
    mesh = plsc.VectorSubcoreMesh(core_axis_name="c", subcore_axis_name="s")
    k = functools.partial(
        pl.kernel,
        mesh=mesh,
        out_type=jax.ShapeDtypeStruct((2, _N, _ACCW), jnp.float32),
        scratch_types=[
            pltpu.VMEM((_B,), jnp.int32),
            pltpu.VMEM((_B,), jnp.int32),
            pltpu.VMEM((_B, 48), jnp.float32),
            pltpu.VMEM((_B, 96), jnp.float32),
            pltpu.VMEM((_B, _ACCW), jnp.float32),
            pltpu.VMEM((256, 16), jnp.float32),
            pltpu.VMEM((128, 16), jnp.float32),
            pltpu.VMEM((144, 16), jnp.float32),
            pltpu.VMEM_SHARED((_N, _ACCW), jnp.float32),
            pltpu.SemaphoreType.DMA,
            pltpu.SemaphoreType.DMA,
        ],
        compiler_params=pltpu.CompilerParams(needs_layout_passes=False,
                                             use_tc_tiling_on_sc=False),
    )(_sc_body)
    return k(src_tab, dst_tab, w1s, us, edge_index, zeros)


def _b16(x):
    return x.astype(jnp.bfloat16).astype(jnp.float32)


def _fold(w_msg, u_dir):
    w1b = _b16(w_msg[:, :_C])
    w2b = _b16(w_msg[:, _C:])
    ub = _b16(u_dir)
    pq = jnp.kron(w2b.T, jnp.eye(3, dtype=jnp.float32))
    w1s = jnp.broadcast_to(w1b.reshape(256, 1), (256, 16))
    us = jnp.broadcast_to(ub.reshape(2, 128, 1), (2, 128, 16))
    return pq, w1s, us


def _finish_stats(stats):
    mean = jnp.sum(stats[:, 0, :], axis=0) / _N
    ex2 = jnp.sum(stats[:, 1, :], axis=0) / _N
    var = ex2 - mean * mean
    return jnp.stack([mean, var])


def kernel(pos, edge_index, W_lift, W_msg_0, U_dir_0, gamma_0,
           W_msg_1, U_dir_1, gamma_1):
    f32 = jnp.float32
    s_mat = jnp.asarray(np.kron(np.eye(16), np.ones((3, 1))), f32)
    st_mat = s_mat.T
    zeros = jnp.zeros((_N, _ACCW), f32)

    pq0, w1s0, us0 = _fold(W_msg_0, U_dir_0)
    lift = jnp.kron(W_lift.reshape(1, _C), jnp.eye(3, dtype=f32))
    src0, dst0 = _tables0(pos, lift, pq0)

    acc_a = _sc_edge_pass(src0, dst0, w1s0, us0, edge_index, zeros)

    mv_a = _finish_stats(_stats(acc_a, s_mat))
    pq1, w1s1, us1 = _fold(W_msg_1, U_dir_1)
    src1, dst1 = _tables(acc_a, mv_a, gamma_0.reshape(1, _C),
                         s_mat, st_mat, pq1)

    acc_b = _sc_edge_pass(src1, dst1, w1s1, us1, edge_index, zeros)

    mv_b = _finish_stats(_stats(acc_b, s_mat))
    return _readout(acc_b, mv_b, gamma_1.reshape(1, _C), s_mat, st_mat)
